# baseline (speedup 1.0000x reference)
import jax
import jax.numpy as jnp
from jax.experimental import pallas as pl
from jax.experimental.pallas import tpu as pltpu

_NC = 3
_NSET = 2


def _plane_kernel(na_ref, row_ref, out_hbm, scratch, sems):
    b = pl.program_id(0)
    nb = pl.num_programs(0)
    sset = jax.lax.rem(b, _NSET)
    na = na_ref[b]

    c3t = row_ref[0, :, 0, :]
    c3 = c3t.T

    n = c3.shape[0]
    jio = jax.lax.broadcasted_iota(jnp.int32, (n, n), 0)
    kio = jax.lax.broadcasted_iota(jnp.int32, (n, n), 1)
    mask = (jio < na) & (kio < na)
    half = n // 2

    @pl.when(b >= _NSET)
    def _wait_prev():
        for c in range(_NC):
            pltpu.make_async_copy(
                scratch.at[sset, c], out_hbm.at[b - _NSET, c],
                sems.at[sset, c]).wait()

    for c in range(_NC):
        col = c3[:, c:c + 1]
        row = c3t[c:c + 1, :]
        scratch[sset, c] = jnp.where(mask, col - row, jnp.float32(0.0))
        pltpu.make_async_copy(scratch.at[sset, c], out_hbm.at[b, c],
                              sems.at[sset, c]).start()

    @pl.when(b == nb - 1)
    def _drain():
        for s in range(_NSET):
            prev = nb - _NSET + s
            for c in range(_NC):
                pltpu.make_async_copy(
                    scratch.at[jax.lax.rem(jnp.int32(prev), _NSET), c],
                    out_hbm.at[prev, c],
                    sems.at[jax.lax.rem(jnp.int32(prev), _NSET), c]).wait()


from jax.experimental.pallas import tpu_sc as plsc
from jax import lax
import functools

_SC_NCORE = 2
_SC_NSUB = 16
_SC_NW = _SC_NCORE * _SC_NSUB


def _sc_body(a_hbm, mj_hbm, out_hbm, a_v, mj_v, obuf):
    cid = lax.axis_index("c")
    sid = lax.axis_index("s")
    wid = sid * _SC_NCORE + cid
    n = 512
    rpw = n // _SC_NW

    def plane_body(pidx, _):
        b = pidx // 3
        c = lax.rem(pidx, 3)
        pltpu.sync_copy(a_hbm.at[b, c], a_v)
        pltpu.sync_copy(mj_hbm.at[b], mj_v)


        xjv = a_v[pl.ds(wid * rpw, rpw)]
        mjv = mj_v[pl.ds(wid * rpw, rpw)]
        for r in range(rpw):
            xj = xjv[r]
            mj = mjv[r]
            for g in range(n // 16):
                av = a_v[pl.ds(g * 16, 16)]
                iv = mj_v[pl.ds(g * 16, 16)]
                obuf[r, pl.ds(g * 16, 16)] = xj * iv - mj * av
        pltpu.sync_copy(obuf, out_hbm.at[b, c, pl.ds(wid * rpw, rpw)])
        return 0

    lax.fori_loop(0, 24, plane_body, 0)


def _kernel_sc(coords, num_atoms):
    bsz, flat = coords.shape
    maxa = flat // 3
    na = num_atoms.astype(jnp.int32)
    x3 = coords.reshape(bsz, maxa, 3).transpose(0, 2, 1)
    ind = (jnp.arange(maxa)[None, :] < na[:, None]).astype(jnp.float32)
    a = x3 * ind[:, None, :]
    rpw = maxa // _SC_NW
    mesh = plsc.VectorSubcoreMesh(core_axis_name="c", subcore_axis_name="s")
    k = functools.partial(
        pl.kernel, mesh=mesh,
        out_type=jax.ShapeDtypeStruct((bsz, 3, maxa, maxa), jnp.float32),
        scratch_types=[
            pltpu.VMEM((maxa,), jnp.float32),
            pltpu.VMEM((maxa,), jnp.float32),
            pltpu.VMEM((rpw, maxa), jnp.float32),
        ],
    )(_sc_body)
    out = k(a, ind)
    return out.transpose(0, 2, 3, 1)


def kernel(coords, num_atoms):
    return _kernel_sc(coords, num_atoms)


def _kernel_tc(coords, num_atoms):
    bsz, flat = coords.shape
    maxa = flat // 3
    na = num_atoms.astype(jnp.int32)
    out = pl.pallas_call(
        _plane_kernel,
        grid_spec=pltpu.PrefetchScalarGridSpec(
            num_scalar_prefetch=1,
            grid=(bsz,),
            in_specs=[
                pl.BlockSpec((1, _NC, 1, maxa), lambda b, na_ref: (b, 0, 0, 0)),
            ],
            out_specs=pl.BlockSpec(memory_space=pl.ANY),
            scratch_shapes=[
                pltpu.VMEM((_NSET, _NC, maxa, maxa), jnp.float32),
                pltpu.SemaphoreType.DMA((_NSET, _NC)),
            ],
        ),
        out_shape=jax.ShapeDtypeStruct((bsz, _NC, maxa, maxa), jnp.float32),
    )(na, coords.reshape(bsz, maxa, 3).transpose(0, 2, 1).reshape(
        bsz, _NC, 1, maxa))
    return out.transpose(0, 2, 3, 1)


# speedup vs baseline: 10.0656x; 10.0656x over previous
import jax
import jax.numpy as jnp
from jax.experimental import pallas as pl
from jax.experimental.pallas import tpu as pltpu

_NC = 3
_NSET = 2


def _plane_kernel(na_ref, row_ref, out_hbm, scratch, sems):
    b = pl.program_id(0)
    nb = pl.num_programs(0)
    sset = jax.lax.rem(b, _NSET)
    na = na_ref[b]

    c3t = row_ref[0, :, 0, :]
    c3 = c3t.T

    n = c3.shape[0]
    jio = jax.lax.broadcasted_iota(jnp.int32, (n, n), 0)
    kio = jax.lax.broadcasted_iota(jnp.int32, (n, n), 1)
    mask = (jio < na) & (kio < na)

    @pl.when(b >= _NSET)
    def _wait_prev():
        for c in range(_NC):
            pltpu.make_async_copy(
                scratch.at[sset, c], out_hbm.at[b - _NSET, c],
                sems.at[sset, c]).wait()

    for c in range(_NC):
        col = c3[:, c:c + 1]
        row = c3t[c:c + 1, :]
        scratch[sset, c] = jnp.where(mask, col - row, jnp.float32(0.0))
        pltpu.make_async_copy(scratch.at[sset, c], out_hbm.at[b, c],
                              sems.at[sset, c]).start()

    @pl.when(b == nb - 1)
    def _drain():
        for s in range(_NSET):
            prev = nb - _NSET + s
            for c in range(_NC):
                pltpu.make_async_copy(
                    scratch.at[jax.lax.rem(jnp.int32(prev), _NSET), c],
                    out_hbm.at[prev, c],
                    sems.at[jax.lax.rem(jnp.int32(prev), _NSET), c]).wait()


def kernel(coords, num_atoms):
    bsz, flat = coords.shape
    maxa = flat // 3
    na = num_atoms.astype(jnp.int32)
    out = pl.pallas_call(
        _plane_kernel,
        grid_spec=pltpu.PrefetchScalarGridSpec(
            num_scalar_prefetch=1,
            grid=(bsz,),
            in_specs=[
                pl.BlockSpec((1, _NC, 1, maxa), lambda b, na_ref: (b, 0, 0, 0)),
            ],
            out_specs=pl.BlockSpec(memory_space=pl.ANY),
            scratch_shapes=[
                pltpu.VMEM((_NSET, _NC, maxa, maxa), jnp.float32),
                pltpu.SemaphoreType.DMA((_NSET, _NC)),
            ],
        ),
        out_shape=jax.ShapeDtypeStruct((bsz, _NC, maxa, maxa), jnp.float32),
    )(na, coords.reshape(bsz, maxa, 3).transpose(0, 2, 1).reshape(
        bsz, _NC, 1, maxa))
    return out.transpose(0, 2, 3, 1)
